# Initial kernel scaffold; baseline (speedup 1.0000x reference)
import jax
import jax.numpy as jnp
from jax.experimental import pallas as pl


def kernel(x, edge_index, edge_weight, W_z, b_z, W_r, b_r, W_h, b_h, W_lin, b_lin):
    raise NotImplementedError("write your pallas kernel here")


# speedup vs baseline: 1.3469x; 1.3469x over previous
import jax
import jax.numpy as jnp
from jax.experimental import pallas as pl
from jax.experimental.pallas import tpu as pltpu

_D_IN = 128
_D_HID = 32
_BLOCK = 2000


def _fused_body(x_ref, wz_ref, wh_ref, bzh_ref, wlin_ref, blin_ref, out_ref):
    x = x_ref[...]
    wz = wz_ref[0, 0, :_D_IN, :] + wz_ref[1, 0, :_D_IN, :]
    wh = wh_ref[0, 0, :_D_IN, :] + wh_ref[1, 0, :_D_IN, :]
    w = jnp.concatenate([wz, wh], axis=1)
    zh = jnp.dot(x, w, preferred_element_type=jnp.float32) + bzh_ref[...]
    z = jax.nn.sigmoid(zh[:, :_D_HID])
    ht = jnp.tanh(zh[:, _D_HID:])
    h = jnp.maximum((1.0 - z) * ht, 0.0)
    out_ref[...] = (
        jnp.dot(h, wlin_ref[...], preferred_element_type=jnp.float32)
        + blin_ref[...]
    )


def kernel(x, edge_index, edge_weight, W_z, b_z, W_r, b_r, W_h, b_h, W_lin, b_lin):
    n = x.shape[0]
    out_len = W_lin.shape[1]
    bzh = jnp.concatenate([b_z, b_h]).reshape(1, 2 * _D_HID)
    blin = b_lin.reshape(1, out_len)
    grid = (n // _BLOCK,)
    return pl.pallas_call(
        _fused_body,
        grid=grid,
        in_specs=[
            pl.BlockSpec((_BLOCK, _D_IN), lambda i: (i, 0)),
            pl.BlockSpec(W_z.shape, lambda i: (0, 0, 0, 0)),
            pl.BlockSpec(W_h.shape, lambda i: (0, 0, 0, 0)),
            pl.BlockSpec((1, 2 * _D_HID), lambda i: (0, 0)),
            pl.BlockSpec(W_lin.shape, lambda i: (0, 0)),
            pl.BlockSpec((1, out_len), lambda i: (0, 0)),
        ],
        out_specs=pl.BlockSpec((_BLOCK, out_len), lambda i: (i, 0)),
        out_shape=jax.ShapeDtypeStruct((n, out_len), jnp.float32),
    )(x, W_z, W_h, bzh, W_lin, blin)


# speedup vs baseline: 1.4484x; 1.0754x over previous
import jax
import jax.numpy as jnp
from jax.experimental import pallas as pl
from jax.experimental.pallas import tpu as pltpu

_D_IN = 128
_D_HID = 32
_BLOCK = 2000


def _fused_body(x_ref, wz_ref, wh_ref, bz_ref, bh_ref, wlin_ref, blin_ref, out_ref):
    x = x_ref[...]
    wz = wz_ref[0, 0, :_D_IN, :] + wz_ref[1, 0, :_D_IN, :]
    wh = wh_ref[0, 0, :_D_IN, :] + wh_ref[1, 0, :_D_IN, :]
    w = jnp.concatenate([wz, wh], axis=1)
    b = jnp.concatenate([bz_ref[...], bh_ref[...]])
    zh = jnp.dot(x, w, preferred_element_type=jnp.float32) + b
    z = jax.nn.sigmoid(zh[:, :_D_HID])
    ht = jnp.tanh(zh[:, _D_HID:])
    h = jnp.maximum((1.0 - z) * ht, 0.0)
    out_ref[...] = (
        jnp.dot(h, wlin_ref[...], preferred_element_type=jnp.float32)
        + blin_ref[...]
    )


def kernel(x, edge_index, edge_weight, W_z, b_z, W_r, b_r, W_h, b_h, W_lin, b_lin):
    n = x.shape[0]
    out_len = W_lin.shape[1]
    grid = (n // _BLOCK,)
    return pl.pallas_call(
        _fused_body,
        grid=grid,
        in_specs=[
            pl.BlockSpec((_BLOCK, _D_IN), lambda i: (i, 0)),
            pl.BlockSpec(W_z.shape, lambda i: (0, 0, 0, 0)),
            pl.BlockSpec(W_h.shape, lambda i: (0, 0, 0, 0)),
            pl.BlockSpec(b_z.shape, lambda i: (0,)),
            pl.BlockSpec(b_h.shape, lambda i: (0,)),
            pl.BlockSpec(W_lin.shape, lambda i: (0, 0)),
            pl.BlockSpec(b_lin.shape, lambda i: (0,)),
        ],
        out_specs=pl.BlockSpec((_BLOCK, out_len), lambda i: (i, 0)),
        out_shape=jax.ShapeDtypeStruct((n, out_len), jnp.float32),
    )(x, W_z, W_h, b_z, b_h, W_lin, b_lin)
